# baseline (speedup 1.0000x reference)
import functools
import jax
import jax.numpy as jnp
from jax import lax
from jax.experimental import pallas as pl
from jax.experimental.pallas import tpu as pltpu, tpu_sc as plsc

_B = 16384
_F = 26
_V = 100000
_D = 64

_NW = 32
_BF = _B * _F
_PER_W = _BF // _NW
_CHUNK = 128
_NCH = _PER_W // _CHUNK
_SCH = 256
_CPS = _SCH // _CHUNK
_NSUP = _PER_W // _SCH
_NBUF = 4

_VB = 512
_HALF = 50176
_NVB = _HALF // _VB


def _tc_body(lo_ref, hi_ref, out_ref):
    lo = lo_ref[0]
    hi = hi_ref[0]
    out_ref[0] = jnp.concatenate([lo.T, hi.T], axis=1)


_tc_relayout = pl.pallas_call(
    _tc_body,
    grid=(_F, _NVB),
    in_specs=[
        pl.BlockSpec((1, _D, _VB), lambda f, v: (f, 0, v)),
        pl.BlockSpec((1, _D, _VB), lambda f, v: (f, 0, _NVB + v)),
    ],
    out_specs=pl.BlockSpec((1, _VB, 2 * _D), lambda f, v: (f, v, 0)),
    out_shape=jax.ShapeDtypeStruct((_F, _HALF, 2 * _D), jnp.float32),
)


def _make_sc_gather():
    mesh = plsc.VectorSubcoreMesh(core_axis_name="c", subcore_axis_name="s")

    @functools.partial(
        pl.kernel,
        mesh=mesh,
        out_type=jax.ShapeDtypeStruct((_NW * _NSUP, _SCH, _D), jnp.float32),
        scratch_types=[
            pltpu.VMEM((_NCH, _CHUNK), jnp.int32),
            pltpu.VMEM((_NBUF, _SCH, _D), jnp.float32),
            [pltpu.SemaphoreType.DMA] * _NBUF,
            [pltpu.SemaphoreType.DMA] * _NBUF,
        ],
        compiler_params=pltpu.CompilerParams(use_tc_tiling_on_sc=False),
    )
    def k(idx_hbm, table_hbm, out_hbm, idx_v, rows_v, gsems, wsems):
        wid = lax.axis_index("s") * 2 + lax.axis_index("c")
        pltpu.sync_copy(idx_hbm.at[wid], idx_v)
        lane = lax.iota(jnp.int32, 16)
        obase = wid * _NSUP

        def fire(S, slot):
            for c in range(_CPS):
                r = S * _CPS + c
                for u in range(_CHUNK // 16):
                    col = u * 16
                    f = lax.rem((r * (_CHUNK // 16) + u) * 16 + lane, _F)
                    v = idx_v[r, pl.ds(col, 16)]
                    h = jnp.where(v >= _HALF, 1, 0)
                    idx_v[r, pl.ds(col, 16)] = (
                        f * (2 * _HALF) + 2 * v - (2 * _HALF - 1) * h)
                pltpu.async_copy(
                    table_hbm.at[idx_v.at[r]],
                    rows_v.at[slot, pl.ds(c * _CHUNK, _CHUNK)],
                    gsems[slot])

        def drain(slot, sem):
            pltpu.make_async_copy(out_hbm.at[0], rows_v.at[slot], sem).wait()

        fire(0, 0)
        fire(1, 1)

        def step(S, slot):
            drain(slot, gsems[slot])
            pltpu.async_copy(rows_v.at[slot], out_hbm.at[obase + S],
                             wsems[slot])
            nslot = (slot + 2) % _NBUF

            @pl.when(S >= 2)
            def _():
                drain(nslot, wsems[nslot])

            @pl.when(S + 2 < _NSUP)
            def _():
                fire(S + 2, nslot)

        def outer(t, carry):
            for b in range(_NBUF):
                step(t * _NBUF + b, b)
            return carry

        lax.fori_loop(0, _NSUP // _NBUF, outer, 0)
        s2 = (_NSUP - 2) % _NBUF
        s1 = (_NSUP - 1) % _NBUF
        drain(s2, wsems[s2])
        drain(s1, wsems[s1])

    return k


_sc_gather = _make_sc_gather()


@jax.jit
def kernel(indices, tables):
    idx3 = indices.reshape(_NW, _NCH, _CHUNK)
    tab_dv = tables.transpose(0, 2, 1)
    t128 = _tc_relayout(tab_dv, tab_dv)
    table_flat = t128.reshape(2 * _F * _HALF, _D)
    out = _sc_gather(idx3, table_flat)
    return out.reshape(_B, _F, _D)


# speedup vs baseline: 2.0428x; 2.0428x over previous
import functools
import jax
import jax.numpy as jnp
from jax import lax
from jax.experimental import pallas as pl
from jax.experimental.pallas import tpu as pltpu, tpu_sc as plsc

_B = 16384
_F = 26
_V = 100000
_D = 64

_NW = 32
_BF = _B * _F
_PER_W = _BF // _NW
_CHUNK = 128
_NCH = _PER_W // _CHUNK
_SCH = 256
_CPS = _SCH // _CHUNK
_NSUP = _PER_W // _SCH
_NBUF = 4

_VB = 3584
_HALF = 50176
_NVB = _HALF // _VB


def _tc_body(lo_ref, hi_ref, out_ref):
    lo = lo_ref[0]
    hi = hi_ref[0]
    out_ref[0] = jnp.concatenate([lo.T, hi.T], axis=1)


_tc_relayout = pl.pallas_call(
    _tc_body,
    grid=(_F, _NVB),
    in_specs=[
        pl.BlockSpec((1, _D, _VB), lambda f, v: (f, 0, v)),
        pl.BlockSpec((1, _D, _VB), lambda f, v: (f, 0, _NVB + v)),
    ],
    out_specs=pl.BlockSpec((1, _VB, 2 * _D), lambda f, v: (f, v, 0)),
    out_shape=jax.ShapeDtypeStruct((_F, _HALF, 2 * _D), jnp.float32),
)


def _make_sc_gather():
    mesh = plsc.VectorSubcoreMesh(core_axis_name="c", subcore_axis_name="s")

    @functools.partial(
        pl.kernel,
        mesh=mesh,
        out_type=jax.ShapeDtypeStruct((_NW * _NSUP, _SCH, _D), jnp.float32),
        scratch_types=[
            pltpu.VMEM((_NCH, _CHUNK), jnp.int32),
            pltpu.VMEM((_NBUF, _SCH, _D), jnp.float32),
            [pltpu.SemaphoreType.DMA] * _NBUF,
            [pltpu.SemaphoreType.DMA] * _NBUF,
        ],
        compiler_params=pltpu.CompilerParams(use_tc_tiling_on_sc=False),
    )
    def k(idx_hbm, table_hbm, out_hbm, idx_v, rows_v, gsems, wsems):
        wid = lax.axis_index("s") * 2 + lax.axis_index("c")
        pltpu.sync_copy(idx_hbm.at[wid], idx_v)
        lane = lax.iota(jnp.int32, 16)
        obase = wid * _NSUP

        def fire(S, slot):
            for c in range(_CPS):
                r = S * _CPS + c
                for u in range(_CHUNK // 16):
                    col = u * 16
                    f = lax.rem((r * (_CHUNK // 16) + u) * 16 + lane, _F)
                    v = idx_v[r, pl.ds(col, 16)]
                    h = jnp.where(v >= _HALF, 1, 0)
                    idx_v[r, pl.ds(col, 16)] = (
                        f * (2 * _HALF) + 2 * v - (2 * _HALF - 1) * h)
                pltpu.async_copy(
                    table_hbm.at[idx_v.at[r]],
                    rows_v.at[slot, pl.ds(c * _CHUNK, _CHUNK)],
                    gsems[slot])

        def drain(slot, sem):
            pltpu.make_async_copy(out_hbm.at[0], rows_v.at[slot], sem).wait()

        fire(0, 0)
        fire(1, 1)

        def step(S, slot):
            drain(slot, gsems[slot])
            pltpu.async_copy(rows_v.at[slot], out_hbm.at[obase + S],
                             wsems[slot])
            nslot = (slot + 2) % _NBUF

            @pl.when(S >= 2)
            def _():
                drain(nslot, wsems[nslot])

            @pl.when(S + 2 < _NSUP)
            def _():
                fire(S + 2, nslot)

        def outer(t, carry):
            for b in range(_NBUF):
                step(t * _NBUF + b, b)
            return carry

        lax.fori_loop(0, _NSUP // _NBUF, outer, 0)
        s2 = (_NSUP - 2) % _NBUF
        s1 = (_NSUP - 1) % _NBUF
        drain(s2, wsems[s2])
        drain(s1, wsems[s1])

    return k


_sc_gather = _make_sc_gather()


@jax.jit
def kernel(indices, tables):
    idx3 = indices.reshape(_NW, _NCH, _CHUNK)
    tab_dv = tables.transpose(0, 2, 1)
    t128 = _tc_relayout(tab_dv, tab_dv)
    table_flat = t128.reshape(2 * _F * _HALF, _D)
    out = _sc_gather(idx3, table_flat)
    return out.reshape(_B, _F, _D)


# speedup vs baseline: 2.2580x; 1.1054x over previous
import functools
import jax
import jax.numpy as jnp
from jax import lax
from jax.experimental import pallas as pl
from jax.experimental.pallas import tpu as pltpu, tpu_sc as plsc

_B = 16384
_F = 26
_V = 100000
_D = 64

_NW = 32
_BF = _B * _F
_PER_W = _BF // _NW
_CHUNK = 128
_NCH = _PER_W // _CHUNK
_SCH = 256
_CPS = _SCH // _CHUNK
_NSUP = _PER_W // _SCH
_NBUF = 4

_VB = 7168
_HALF = 50176
_NVB = _HALF // _VB


def _tc_body(lo_ref, hi_ref, out_ref):
    lo = lo_ref[0]
    hi = hi_ref[0]
    out_ref[0] = jnp.concatenate([lo.T, hi.T], axis=1)


_tc_relayout = pl.pallas_call(
    _tc_body,
    grid=(_F, _NVB),
    in_specs=[
        pl.BlockSpec((1, _D, _VB), lambda f, v: (f, 0, v)),
        pl.BlockSpec((1, _D, _VB), lambda f, v: (f, 0, _NVB + v)),
    ],
    out_specs=pl.BlockSpec((1, _VB, 2 * _D), lambda f, v: (f, v, 0)),
    out_shape=jax.ShapeDtypeStruct((_F, _HALF, 2 * _D), jnp.float32),
)


def _make_sc_gather():
    mesh = plsc.VectorSubcoreMesh(core_axis_name="c", subcore_axis_name="s")

    @functools.partial(
        pl.kernel,
        mesh=mesh,
        out_type=jax.ShapeDtypeStruct((_NW * _NSUP, _SCH, _D), jnp.float32),
        scratch_types=[
            pltpu.VMEM((_NCH, _CHUNK), jnp.int32),
            pltpu.VMEM((_NBUF, _SCH, _D), jnp.float32),
            [pltpu.SemaphoreType.DMA] * _NBUF,
            [pltpu.SemaphoreType.DMA] * _NBUF,
        ],
        compiler_params=pltpu.CompilerParams(use_tc_tiling_on_sc=False),
    )
    def k(idx_hbm, table_hbm, out_hbm, idx_v, rows_v, gsems, wsems):
        wid = lax.axis_index("s") * 2 + lax.axis_index("c")
        pltpu.sync_copy(idx_hbm.at[wid], idx_v)
        lane = lax.iota(jnp.int32, 16)
        obase = wid * _NSUP

        def fire(S, slot):
            for c in range(_CPS):
                r = S * _CPS + c
                for u in range(_CHUNK // 16):
                    col = u * 16
                    f = lax.rem((r * (_CHUNK // 16) + u) * 16 + lane, _F)
                    v = idx_v[r, pl.ds(col, 16)]
                    h = jnp.where(v >= _HALF, 1, 0)
                    idx_v[r, pl.ds(col, 16)] = (
                        f * (2 * _HALF) + 2 * v - (2 * _HALF - 1) * h)
                pltpu.async_copy(
                    table_hbm.at[idx_v.at[r]],
                    rows_v.at[slot, pl.ds(c * _CHUNK, _CHUNK)],
                    gsems[slot])

        def drain(slot, sem):
            pltpu.make_async_copy(out_hbm.at[0], rows_v.at[slot], sem).wait()

        fire(0, 0)
        fire(1, 1)

        def step(S, slot):
            drain(slot, gsems[slot])
            pltpu.async_copy(rows_v.at[slot], out_hbm.at[obase + S],
                             wsems[slot])
            nslot = (slot + 2) % _NBUF

            @pl.when(S >= 2)
            def _():
                drain(nslot, wsems[nslot])

            @pl.when(S + 2 < _NSUP)
            def _():
                fire(S + 2, nslot)

        def outer(t, carry):
            for b in range(_NBUF):
                step(t * _NBUF + b, b)
            return carry

        lax.fori_loop(0, _NSUP // _NBUF, outer, 0)
        s2 = (_NSUP - 2) % _NBUF
        s1 = (_NSUP - 1) % _NBUF
        drain(s2, wsems[s2])
        drain(s1, wsems[s1])

    return k


_sc_gather = _make_sc_gather()


@jax.jit
def kernel(indices, tables):
    idx3 = indices.reshape(_NW, _NCH, _CHUNK)
    tab_dv = tables.transpose(0, 2, 1)
    t128 = _tc_relayout(tab_dv, tab_dv)
    table_flat = t128.reshape(2 * _F * _HALF, _D)
    out = _sc_gather(idx3, table_flat)
    return out.reshape(_B, _F, _D)


# speedup vs baseline: 2.3812x; 1.0546x over previous
import functools
import jax
import jax.numpy as jnp
from jax import lax
from jax.experimental import pallas as pl
from jax.experimental.pallas import tpu as pltpu, tpu_sc as plsc

_B = 16384
_F = 26
_V = 100000
_D = 64

_NW = 32
_BF = _B * _F
_PER_W = _BF // _NW
_CHUNK = 128
_NCH = _PER_W // _CHUNK
_SCH = 256
_CPS = _SCH // _CHUNK
_NSUP = _PER_W // _SCH
_NBUF = 4

_VB = 7168
_HALF = 50176
_NVB = _HALF // _VB
_J = _F // 2
_BB = 2048


def _tc_body(lo_ref, hi_ref, out_ref):
    lo = lo_ref[0]
    hi = hi_ref[0]
    out_ref[0] = jnp.concatenate([lo.T, hi.T], axis=1)


_tc_relayout = pl.pallas_call(
    _tc_body,
    grid=(_F, _NVB),
    in_specs=[
        pl.BlockSpec((1, _D, _VB), lambda f, v: (f, 0, v)),
        pl.BlockSpec((1, _D, _VB), lambda f, v: (f, 0, _NVB + v)),
    ],
    out_specs=pl.BlockSpec((1, _VB, 2 * _D), lambda f, v: (f, v, 0)),
    out_shape=jax.ShapeDtypeStruct((_F, _HALF, 2 * _D), jnp.float32),
)


def _tc_out_body(in_ref, out_ref):
    out_ref[0] = in_ref[0].T


_tc_out = pl.pallas_call(
    _tc_out_body,
    grid=(_J, _B // _BB),
    in_specs=[pl.BlockSpec((1, _BB, 2 * _D), lambda j, b: (j, b, 0))],
    out_specs=pl.BlockSpec((1, 2 * _D, _BB), lambda j, b: (j, 0, b)),
    out_shape=jax.ShapeDtypeStruct((_J, 2 * _D, _B), jnp.float32),
)


def _make_sc_gather():
    mesh = plsc.VectorSubcoreMesh(core_axis_name="c", subcore_axis_name="s")

    @functools.partial(
        pl.kernel,
        mesh=mesh,
        out_type=jax.ShapeDtypeStruct((_NW * _NSUP, _SCH, _D), jnp.float32),
        scratch_types=[
            pltpu.VMEM((_NCH, _CHUNK), jnp.int32),
            pltpu.VMEM((_NBUF, _SCH, _D), jnp.float32),
            [pltpu.SemaphoreType.DMA] * _NBUF,
            [pltpu.SemaphoreType.DMA] * _NBUF,
        ],
        compiler_params=pltpu.CompilerParams(use_tc_tiling_on_sc=False),
    )
    def k(idx_hbm, table_hbm, out_hbm, idx_v, rows_v, gsems, wsems):
        wid = lax.axis_index("s") * 2 + lax.axis_index("c")
        pltpu.sync_copy(idx_hbm.at[wid], idx_v)
        lane = lax.iota(jnp.int32, 16)
        obase = wid * _NSUP
        nbase = wid * _PER_W

        def fire(S, slot):
            for c in range(_CPS):
                r = S * _CPS + c
                for u in range(_CHUNK // 16):
                    col = u * 16
                    n = nbase + r * _CHUNK + col + lane
                    f = 2 * (n >> 15) + (n & 1)
                    v = idx_v[r, pl.ds(col, 16)]
                    h = jnp.where(v >= _HALF, 1, 0)
                    idx_v[r, pl.ds(col, 16)] = (
                        f * (2 * _HALF) + 2 * v - (2 * _HALF - 1) * h)
                pltpu.async_copy(
                    table_hbm.at[idx_v.at[r]],
                    rows_v.at[slot, pl.ds(c * _CHUNK, _CHUNK)],
                    gsems[slot])

        def drain(slot, sem):
            pltpu.make_async_copy(out_hbm.at[0], rows_v.at[slot], sem).wait()

        fire(0, 0)
        fire(1, 1)

        def step(S, slot):
            drain(slot, gsems[slot])
            pltpu.async_copy(rows_v.at[slot], out_hbm.at[obase + S],
                             wsems[slot])
            nslot = (slot + 2) % _NBUF

            @pl.when(S >= 2)
            def _():
                drain(nslot, wsems[nslot])

            @pl.when(S + 2 < _NSUP)
            def _():
                fire(S + 2, nslot)

        def outer(t, carry):
            for b in range(_NBUF):
                step(t * _NBUF + b, b)
            return carry

        lax.fori_loop(0, _NSUP // _NBUF, outer, 0)
        s2 = (_NSUP - 2) % _NBUF
        s1 = (_NSUP - 1) % _NBUF
        drain(s2, wsems[s2])
        drain(s1, wsems[s1])

    return k


_sc_gather = _make_sc_gather()


@jax.jit
def kernel(indices, tables):
    idx3 = (indices.reshape(_B, _J, 2).transpose(1, 0, 2)
            .reshape(_NW, _NCH, _CHUNK))
    tab_dv = tables.transpose(0, 2, 1)
    t128 = _tc_relayout(tab_dv, tab_dv)
    table_flat = t128.reshape(2 * _F * _HALF, _D)
    out = _sc_gather(idx3, table_flat)
    out_fdb = _tc_out(out.reshape(_J, _B, 2 * _D))
    return (out_fdb.reshape(_J, 2, _D, _B)
            .transpose(3, 0, 1, 2).reshape(_B, _F, _D))


# speedup vs baseline: 2.5888x; 1.0872x over previous
import functools
import jax
import jax.numpy as jnp
from jax import lax
from jax.experimental import pallas as pl
from jax.experimental.pallas import tpu as pltpu, tpu_sc as plsc

_B = 16384
_F = 26
_V = 100000
_D = 64

_NW = 32
_BF = _B * _F
_PER_W = _BF // _NW
_CHUNK = 128
_NCH = _PER_W // _CHUNK
_SCH = 256
_CPS = _SCH // _CHUNK
_NSUP = _PER_W // _SCH
_NBUF = 4

_VB = 4096
_NVB = -(-_V // _VB)
_J = _F // 2
_BB = 2048


def _tc_body(in_ref, out_ref):
    x = in_ref[...].reshape(2 * _D, _VB)
    out_ref[0] = x.T


_tc_relayout = pl.pallas_call(
    _tc_body,
    grid=(_J, _NVB),
    in_specs=[pl.BlockSpec((2, _D, _VB), lambda j, v: (j, 0, v))],
    out_specs=pl.BlockSpec((1, _VB, 2 * _D), lambda j, v: (j, v, 0)),
    out_shape=jax.ShapeDtypeStruct((_J, _V, 2 * _D), jnp.float32),
)


def _tc_out_body(in_ref, out_ref):
    out_ref[0] = in_ref[0].T


_tc_out = pl.pallas_call(
    _tc_out_body,
    grid=(_J, _B // _BB),
    in_specs=[pl.BlockSpec((1, _BB, 2 * _D), lambda j, b: (j, b, 0))],
    out_specs=pl.BlockSpec((1, 2 * _D, _BB), lambda j, b: (j, 0, b)),
    out_shape=jax.ShapeDtypeStruct((_J, 2 * _D, _B), jnp.float32),
)


def _make_sc_gather():
    mesh = plsc.VectorSubcoreMesh(core_axis_name="c", subcore_axis_name="s")

    @functools.partial(
        pl.kernel,
        mesh=mesh,
        out_type=jax.ShapeDtypeStruct((_NW * _NSUP, _SCH, _D), jnp.float32),
        scratch_types=[
            pltpu.VMEM((_NCH, _CHUNK), jnp.int32),
            pltpu.VMEM((_NBUF, _SCH, _D), jnp.float32),
            [pltpu.SemaphoreType.DMA] * _NBUF,
            [pltpu.SemaphoreType.DMA] * _NBUF,
        ],
        compiler_params=pltpu.CompilerParams(use_tc_tiling_on_sc=False),
    )
    def k(idx_hbm, table_hbm, out_hbm, idx_v, rows_v, gsems, wsems):
        wid = lax.axis_index("s") * 2 + lax.axis_index("c")
        pltpu.sync_copy(idx_hbm.at[wid], idx_v)
        lane = lax.iota(jnp.int32, 16)
        obase = wid * _NSUP
        nbase = wid * _PER_W

        def fire(S, slot):
            for c in range(_CPS):
                r = S * _CPS + c
                for u in range(_CHUNK // 16):
                    col = u * 16
                    n = nbase + r * _CHUNK + col + lane
                    v = idx_v[r, pl.ds(col, 16)]
                    idx_v[r, pl.ds(col, 16)] = (
                        (n >> 15) * (2 * _V) + 2 * v + (n & 1))
                pltpu.async_copy(
                    table_hbm.at[idx_v.at[r]],
                    rows_v.at[slot, pl.ds(c * _CHUNK, _CHUNK)],
                    gsems[slot])

        def drain(slot, sem):
            pltpu.make_async_copy(out_hbm.at[0], rows_v.at[slot], sem).wait()

        fire(0, 0)
        fire(1, 1)

        def step(S, slot):
            drain(slot, gsems[slot])
            pltpu.async_copy(rows_v.at[slot], out_hbm.at[obase + S],
                             wsems[slot])
            nslot = (slot + 2) % _NBUF

            @pl.when(S >= 2)
            def _():
                drain(nslot, wsems[nslot])

            @pl.when(S + 2 < _NSUP)
            def _():
                fire(S + 2, nslot)

        def outer(t, carry):
            for b in range(_NBUF):
                step(t * _NBUF + b, b)
            return carry

        lax.fori_loop(0, _NSUP // _NBUF, outer, 0)
        s2 = (_NSUP - 2) % _NBUF
        s1 = (_NSUP - 1) % _NBUF
        drain(s2, wsems[s2])
        drain(s1, wsems[s1])

    return k


_sc_gather = _make_sc_gather()


@jax.jit
def kernel(indices, tables):
    idx3 = (indices.reshape(_B, _J, 2).transpose(1, 0, 2)
            .reshape(_NW, _NCH, _CHUNK))
    tab_dv = tables.transpose(0, 2, 1)
    t128 = _tc_relayout(tab_dv)
    table_flat = t128.reshape(2 * _J * _V, _D)
    out = _sc_gather(idx3, table_flat)
    out_fdb = _tc_out(out.reshape(_J, _B, 2 * _D))
    return (out_fdb.reshape(_J, 2, _D, _B)
            .transpose(3, 0, 1, 2).reshape(_B, _F, _D))


# speedup vs baseline: 2.8183x; 1.0886x over previous
import functools
import jax
import jax.numpy as jnp
from jax import lax
from jax.experimental import pallas as pl
from jax.experimental.pallas import tpu as pltpu, tpu_sc as plsc

_B = 16384
_F = 26
_V = 100000
_D = 64

_NW = 32
_BF = _B * _F
_PER_W = _BF // _NW
_CHUNK = 128
_NCH = _PER_W // _CHUNK
_SCH = 256
_CPS = _SCH // _CHUNK
_NSUP = _PER_W // _SCH
_NBUF = 4

_VB = 8192
_NVB = -(-_V // _VB)
_J = _F // 2
_BB = 2048


def _tc_body(in_ref, out_ref):
    x = in_ref[...].reshape(2 * _D, _VB)
    out_ref[0] = x.T


_tc_relayout = pl.pallas_call(
    _tc_body,
    grid=(_J, _NVB),
    in_specs=[pl.BlockSpec((2, _D, _VB), lambda j, v: (j, 0, v))],
    out_specs=pl.BlockSpec((1, _VB, 2 * _D), lambda j, v: (j, v, 0)),
    out_shape=jax.ShapeDtypeStruct((_J, _V, 2 * _D), jnp.float32),
)


def _tc_out_body(in_ref, out_ref):
    out_ref[0] = in_ref[0].T


_tc_out = pl.pallas_call(
    _tc_out_body,
    grid=(_J, _B // _BB),
    in_specs=[pl.BlockSpec((1, _BB, 2 * _D), lambda j, b: (j, b, 0))],
    out_specs=pl.BlockSpec((1, 2 * _D, _BB), lambda j, b: (j, 0, b)),
    out_shape=jax.ShapeDtypeStruct((_J, 2 * _D, _B), jnp.float32),
)


def _make_sc_gather():
    mesh = plsc.VectorSubcoreMesh(core_axis_name="c", subcore_axis_name="s")

    @functools.partial(
        pl.kernel,
        mesh=mesh,
        out_type=jax.ShapeDtypeStruct((_NW * _NSUP, _SCH, _D), jnp.float32),
        scratch_types=[
            pltpu.VMEM((_NCH, _CHUNK), jnp.int32),
            pltpu.VMEM((_NBUF, _SCH, _D), jnp.float32),
            [pltpu.SemaphoreType.DMA] * _NBUF,
            [pltpu.SemaphoreType.DMA] * _NBUF,
        ],
        compiler_params=pltpu.CompilerParams(use_tc_tiling_on_sc=False),
    )
    def k(idx_hbm, table_hbm, out_hbm, idx_v, rows_v, gsems, wsems):
        wid = lax.axis_index("s") * 2 + lax.axis_index("c")
        pltpu.sync_copy(idx_hbm.at[wid], idx_v)
        lane = lax.iota(jnp.int32, 16)
        obase = wid * _NSUP
        nbase = wid * _PER_W

        def fire(S, slot):
            for c in range(_CPS):
                r = S * _CPS + c
                for u in range(_CHUNK // 16):
                    col = u * 16
                    n = nbase + r * _CHUNK + col + lane
                    v = idx_v[r, pl.ds(col, 16)]
                    idx_v[r, pl.ds(col, 16)] = (
                        (n >> 15) * (2 * _V) + 2 * v + (n & 1))
                pltpu.async_copy(
                    table_hbm.at[idx_v.at[r]],
                    rows_v.at[slot, pl.ds(c * _CHUNK, _CHUNK)],
                    gsems[slot])

        def drain(slot, sem):
            pltpu.make_async_copy(out_hbm.at[0], rows_v.at[slot], sem).wait()

        fire(0, 0)
        fire(1, 1)

        def step(S, slot):
            drain(slot, gsems[slot])
            pltpu.async_copy(rows_v.at[slot], out_hbm.at[obase + S],
                             wsems[slot])
            nslot = (slot + 2) % _NBUF

            @pl.when(S >= 2)
            def _():
                drain(nslot, wsems[nslot])

            @pl.when(S + 2 < _NSUP)
            def _():
                fire(S + 2, nslot)

        def outer(t, carry):
            for b in range(_NBUF):
                step(t * _NBUF + b, b)
            return carry

        lax.fori_loop(0, _NSUP // _NBUF, outer, 0)
        s2 = (_NSUP - 2) % _NBUF
        s1 = (_NSUP - 1) % _NBUF
        drain(s2, wsems[s2])
        drain(s1, wsems[s1])

    return k


_sc_gather = _make_sc_gather()


@jax.jit
def kernel(indices, tables):
    idx3 = (indices.reshape(_B, _J, 2).transpose(1, 0, 2)
            .reshape(_NW, _NCH, _CHUNK))
    tab_dv = tables.transpose(0, 2, 1)
    t128 = _tc_relayout(tab_dv)
    table_flat = t128.reshape(2 * _J * _V, _D)
    out = _sc_gather(idx3, table_flat)
    out_fdb = _tc_out(out.reshape(_J, _B, 2 * _D))
    return (out_fdb.reshape(_J, 2, _D, _B)
            .transpose(3, 0, 1, 2).reshape(_B, _F, _D))
